# Initial kernel scaffold; baseline (speedup 1.0000x reference)
import jax
import jax.numpy as jnp
from jax.experimental import pallas as pl


def kernel(seq, W_region, W_word):
    raise NotImplementedError("write your pallas kernel here")


# speedup vs baseline: 3.4751x; 3.4751x over previous
import functools

import jax
import jax.numpy as jnp
from jax import lax
from jax.experimental import pallas as pl
from jax.experimental.pallas import tpu as pltpu
from jax.experimental.pallas import tpu_sc as plsc

VOCAB = 100000
EMB = 32
WIN = 5
B = 1024
L = 200
NWIN = L - WIN + 1
HALF = 112
NH = 2
PW = HALF * NH
SEQ_PAD = 240
NC = 2
NS = 16
NW = NC * NS
ROWS_PER = B // NW
LANES = 16


def _sc_body(seq_hbm, wr_hbm, ww_hbm, out_hbm,
             seq_v, ridx_v, widx_v, reg_v, word_v, out_v, sem):
    wid = lax.axis_index("s") * NC + lax.axis_index("c")
    base = wid * ROWS_PER

    def do_row(t, carry):
        row = base + t
        pltpu.sync_copy(seq_hbm.at[row], seq_v)

        for i in range(WIN):
            for h in range(NH):
                for c in range(HALF // LANES):
                    off = i + h * HALF + c * LANES
                    ridx_v[i, h, pl.ds(c * LANES, LANES)] = (
                        seq_v[pl.ds(off, LANES)] + i * VOCAB)
        for h in range(NH):
            for c in range(HALF // LANES):
                off = (WIN // 2) + h * HALF + c * LANES
                widx_v[h, pl.ds(c * LANES, LANES)] = seq_v[pl.ds(off, LANES)]

        for i in range(WIN):
            for h in range(NH):
                pltpu.async_copy(wr_hbm.at[ridx_v.at[i, h]], reg_v.at[i, h], sem)
        for h in range(NH):
            pltpu.async_copy(ww_hbm.at[widx_v.at[h]], word_v.at[h], sem)
        for i in range(WIN):
            for h in range(NH):
                pltpu.make_async_copy(wr_hbm.at[ridx_v.at[i, h]],
                                      reg_v.at[i, h], sem).wait()
        for h in range(NH):
            pltpu.make_async_copy(ww_hbm.at[widx_v.at[h]],
                                  word_v.at[h], sem).wait()

        for h in range(NH):
            n = HALF if h == 0 else NWIN - HALF

            def comp(j, c, h=h):
                w0 = word_v[h, j, pl.ds(0, LANES)]
                w1 = word_v[h, j, pl.ds(LANES, LANES)]
                a0 = reg_v[0, h, j, pl.ds(0, LANES)] * w0
                a1 = reg_v[0, h, j, pl.ds(LANES, LANES)] * w1
                for i in range(1, WIN):
                    a0 = jnp.maximum(a0, reg_v[i, h, j, pl.ds(0, LANES)] * w0)
                    a1 = jnp.maximum(a1, reg_v[i, h, j, pl.ds(LANES, LANES)] * w1)
                out_v[h, j, pl.ds(0, LANES)] = a0
                out_v[h, j, pl.ds(LANES, LANES)] = a1
                return c

            lax.fori_loop(0, n, comp, 0)

        pltpu.sync_copy(out_v.at[0], out_hbm.at[row, pl.ds(0, HALF)])
        pltpu.sync_copy(out_v.at[1, pl.ds(0, NWIN - HALF)],
                        out_hbm.at[row, pl.ds(HALF, NWIN - HALF)])
        return carry

    lax.fori_loop(0, ROWS_PER, do_row, 0)


@jax.jit
def _run(seq_pad, w_region, w_word):
    mesh = plsc.VectorSubcoreMesh(core_axis_name="c", subcore_axis_name="s",
                                  num_cores=NC, num_subcores=NS)
    return pl.kernel(
        _sc_body,
        out_type=jax.ShapeDtypeStruct((B, NWIN, EMB), jnp.float32),
        mesh=mesh,
        scratch_types=[
            pltpu.VMEM((SEQ_PAD,), jnp.int32),
            pltpu.VMEM((WIN, NH, HALF), jnp.int32),
            pltpu.VMEM((NH, HALF), jnp.int32),
            pltpu.VMEM((WIN, NH, HALF, EMB), jnp.float32),
            pltpu.VMEM((NH, HALF, EMB), jnp.float32),
            pltpu.VMEM((NH, HALF, EMB), jnp.float32),
            pltpu.SemaphoreType.DMA,
        ],
        compiler_params=pltpu.CompilerParams(use_tc_tiling_on_sc=False),
    )(seq_pad, w_region, w_word)


def kernel(seq, W_region, W_word):
    seq_pad = jnp.pad(seq.astype(jnp.int32), ((0, 0), (0, SEQ_PAD - L)))
    return _run(seq_pad, W_region, W_word)
